# baseline (speedup 1.0000x reference)
import jax
import jax.numpy as jnp
from jax import lax
from jax.experimental import pallas as pl
from jax.experimental.pallas import tpu as pltpu
from jax.experimental.pallas import tpu_sc as plsc

NC = 2
NS = 16
NW = NC * NS
DEGW = 128
K = 128
BPW = 80
KA = 80


def _make_sc_agg(n_pad, d, n_edges):
    mesh = plsc.VectorSubcoreMesh(core_axis_name="c", subcore_axis_name="s")
    epw = n_edges // NW
    nblk = epw // KA
    half = nblk // 2

    out_type = jax.ShapeDtypeStruct((NC * n_pad, d), jnp.float32)
    scratch = [
        pltpu.VMEM((KA,), jnp.int32),
        pltpu.VMEM((KA,), jnp.int32),
        pltpu.VMEM((KA,), jnp.int32),
        pltpu.VMEM((KA,), jnp.int32),
        pltpu.VMEM((KA, d), jnp.float32),
        pltpu.VMEM((KA, d), jnp.float32),
        pltpu.VMEM_SHARED((n_pad, d), jnp.float32),
        pltpu.SemaphoreType.DMA,
        pltpu.SemaphoreType.DMA,
        pltpu.SemaphoreType.DMA,
        pltpu.SemaphoreType.DMA,
    ]

    def body(x_hbm, src_hbm, dst_hbm, z_hbm, agg_out,
             src_v1, dst_v1, src_v2, dst_v2, rows_v1, rows_v2, agg_s,
             sem_g1, sem_g2, sem_s1, sem_s2):
        ci = lax.axis_index("c")
        si = lax.axis_index("s")
        wid = ci * NS + si

        @pl.when(si == 0)
        def _init():
            pltpu.sync_copy(z_hbm, agg_s)

        plsc.subcore_barrier()

        base = wid * epw

        @pl.loop(0, half)
        def _blk(i):
            off1 = base + i * KA
            off2 = base + (half + i) * KA
            pltpu.sync_copy(src_hbm.at[pl.ds(off1, KA)], src_v1)
            pltpu.sync_copy(dst_hbm.at[pl.ds(off1, KA)], dst_v1)
            g1 = pltpu.async_copy(x_hbm.at[src_v1], rows_v1, sem_g1)
            pltpu.sync_copy(src_hbm.at[pl.ds(off2, KA)], src_v2)
            pltpu.sync_copy(dst_hbm.at[pl.ds(off2, KA)], dst_v2)
            g2 = pltpu.async_copy(x_hbm.at[src_v2], rows_v2, sem_g2)
            g1.wait()
            s1 = pltpu.async_copy(rows_v1, agg_s.at[dst_v1], sem_s1,
                                  add=True)
            g2.wait()
            s2 = pltpu.async_copy(rows_v2, agg_s.at[dst_v2], sem_s2,
                                  add=True)
            s1.wait()
            s2.wait()

        plsc.subcore_barrier()

        @pl.when(si == 0)
        def _drain():
            pltpu.sync_copy(agg_s, agg_out.at[pl.ds(ci * n_pad, n_pad), :])

    return pl.kernel(body, out_type=out_type, mesh=mesh, scratch_types=scratch)


def _make_sc_deg(n_pad):
    mesh = plsc.VectorSubcoreMesh(core_axis_name="c", subcore_axis_name="s")

    out_type = jax.ShapeDtypeStruct((NC * n_pad, DEGW), jnp.float32)
    scratch = [
        pltpu.VMEM((BPW, K), jnp.int32),
        pltpu.VMEM((K, DEGW), jnp.float32),
        pltpu.VMEM_SHARED((n_pad, DEGW), jnp.float32),
    ]

    def body(dst_hbm, zd_hbm, ones_hbm, deg_out, dst_all, ones_v, deg_s):
        ci = lax.axis_index("c")
        si = lax.axis_index("s")
        wid = ci * NS + si

        @pl.when(si == 0)
        def _init():
            pltpu.sync_copy(zd_hbm, deg_s)

        pltpu.sync_copy(ones_hbm, ones_v)
        pltpu.sync_copy(dst_hbm.at[pl.ds(wid * BPW, BPW), :], dst_all)

        plsc.subcore_barrier()

        @pl.loop(0, BPW)
        def _blk(i):
            pltpu.sync_copy(ones_v, deg_s.at[dst_all.at[i]], add=True)

        plsc.subcore_barrier()

        @pl.when(si == 0)
        def _drain():
            pltpu.sync_copy(deg_s, deg_out.at[pl.ds(ci * n_pad, n_pad), :])

    return pl.kernel(body, out_type=out_type, mesh=mesh, scratch_types=scratch)


def _tc_layer1(agg, deg, x, Wl, Wr, b):
    n, d = x.shape
    n_pad = agg.shape[0] // NC

    def body(agg_ref, deg_ref, x_ref, wl_ref, wr_ref, b_ref, h_ref):
        s = agg_ref[:n, :] + agg_ref[n_pad:n_pad + n, :]
        dv = deg_ref[:n, :] + deg_ref[n_pad:n_pad + n, :]
        mean = s / jnp.maximum(dv[:, :1], 1.0)
        acc = jnp.dot(mean, wl_ref[...], preferred_element_type=jnp.float32)
        acc = acc + jnp.dot(x_ref[...], wr_ref[...],
                            preferred_element_type=jnp.float32)
        acc = acc + b_ref[...]
        h_ref[...] = jnp.maximum(acc, 0.0)

    return pl.pallas_call(
        body, out_shape=jax.ShapeDtypeStruct((n, d), jnp.float32),
    )(agg, deg, x, Wl, Wr, b.reshape(1, d))


def _tc_layer2(agg, deg, h, Wl, Wr, b):
    n, d = h.shape
    n_pad = agg.shape[0] // NC

    def body(agg_ref, deg_ref, h_ref, wl_ref, wr_ref, b_ref, o_ref, ls_ref):
        s = agg_ref[:n, :] + agg_ref[n_pad:n_pad + n, :]
        dv = deg_ref[:n, :] + deg_ref[n_pad:n_pad + n, :]
        mean = s / jnp.maximum(dv[:, :1], 1.0)
        o = jnp.dot(mean, wl_ref[...], preferred_element_type=jnp.float32)
        o = o + jnp.dot(h_ref[...], wr_ref[...],
                        preferred_element_type=jnp.float32)
        o = o + b_ref[...]
        o_ref[...] = o
        m = jnp.max(o, axis=1, keepdims=True)
        lse = jnp.log(jnp.sum(jnp.exp(o - m), axis=1, keepdims=True)) + m
        ls_ref[...] = o - lse

    return pl.pallas_call(
        body,
        out_shape=(jax.ShapeDtypeStruct((n, d), jnp.float32),
                   jax.ShapeDtypeStruct((n, d), jnp.float32)),
    )(agg, deg, h, Wl, Wr, b.reshape(1, d))


def kernel(x, edge_index, W1l, W1r, b1, W2l, W2r, b2):
    n, d = x.shape
    e = edge_index.shape[1]
    ei = edge_index.astype(jnp.int32)
    src, dst = ei[0], ei[1]

    n_pad = -(-n // (NS * 8)) * (NS * 8)
    zeros = jnp.zeros((n_pad, d), jnp.float32)
    zeros_deg = jnp.zeros((n_pad, DEGW), jnp.float32)
    ones = jnp.ones((K, DEGW), jnp.float32)

    e_pad = NW * BPW * K
    pad = e_pad - e
    src_p = jnp.concatenate([src, jnp.zeros((pad,), jnp.int32)])
    dst_p = jnp.concatenate(
        [dst, n + (jnp.arange(pad, dtype=jnp.int32) % (n_pad - n))])
    src2 = src_p.reshape(NW * BPW, K)
    dst2 = dst_p.reshape(NW * BPW, K)

    ea_pad = NW * ((-(-e // (NW * KA)) + 1) // 2 * 2) * KA
    pa = ea_pad - e
    src_a = jnp.concatenate([src, jnp.zeros((pa,), jnp.int32)])
    dst_a = jnp.concatenate(
        [dst, n + (jnp.arange(pa, dtype=jnp.int32) % (n_pad - n))])

    deg = _make_sc_deg(n_pad)(dst2, zeros_deg, ones)
    agg1 = _make_sc_agg(n_pad, d, ea_pad)(x, src_a, dst_a, zeros)
    h = _tc_layer1(agg1, deg, x, W1l, W1r, b1)
    agg2 = _make_sc_agg(n_pad, d, ea_pad)(h, src_a, dst_a, zeros)
    out, ls = _tc_layer2(agg2, deg, h, W2l, W2r, b2)
    return (out, ls)


# speedup vs baseline: 1.1274x; 1.1274x over previous
import dataclasses

import jax
import jax.numpy as jnp
from jax import lax
from jax.experimental import pallas as pl
from jax.experimental.pallas import tpu as pltpu
from jax.experimental.pallas import tpu_sc as plsc

NC = 2
NS = 16
NW = NC * NS
KA = 80


def _make_sc_agg(n_pad, d, n_edges, with_deg=False):
    mesh = plsc.VectorSubcoreMesh(core_axis_name="c", subcore_axis_name="s")
    epw = n_edges // NW
    nblk = epw // KA
    nrow = -(-n_pad // 128)
    nrow8 = -(-nrow // 8) * 8

    out_type = [jax.ShapeDtypeStruct((NC * n_pad, d), jnp.float32)]
    scratch = [
        pltpu.VMEM((KA,), jnp.int32),
        pltpu.VMEM((KA,), jnp.int32),
        pltpu.VMEM((KA, d), jnp.float32),
        pltpu.VMEM_SHARED((n_pad, d), jnp.float32),
        pltpu.SemaphoreType.DMA,
    ]
    if with_deg:
        out_type.append(
            jax.ShapeDtypeStruct((NC * nrow8, 128), jnp.float32))
        scratch += [
            pltpu.VMEM((nrow8, 128), jnp.float32),
            pltpu.VMEM((nrow8,), jnp.int32),
            pltpu.VMEM_SHARED((nrow8, 128), jnp.float32),
        ]

    def body(*refs):
        if with_deg:
            (x_hbm, src_hbm, dst_hbm, z_hbm, iota_hbm, agg_out, deg_out,
             src_v, dst_v, rows_v, agg_s, sem, deg_t, idr_v, deg_s) = refs
        else:
            (x_hbm, src_hbm, dst_hbm, z_hbm, agg_out,
             src_v, dst_v, rows_v, agg_s, sem) = refs
        ci = lax.axis_index("c")
        si = lax.axis_index("s")
        wid = ci * NS + si

        @pl.when(si == 0)
        def _init():
            pltpu.sync_copy(z_hbm, agg_s)
            if with_deg:
                pltpu.sync_copy(z_hbm.at[pl.ds(0, nrow8), :], deg_s)

        if with_deg:
            pltpu.sync_copy(z_hbm.at[pl.ds(0, nrow8), :], deg_t)
            pltpu.sync_copy(iota_hbm, idr_v)

        plsc.subcore_barrier()

        base = wid * epw

        @pl.loop(0, nblk)
        def _blk(i):
            off = base + i * KA
            pltpu.sync_copy(src_hbm.at[pl.ds(off, KA)], src_v)
            pltpu.sync_copy(dst_hbm.at[pl.ds(off, KA)], dst_v)
            gat = pltpu.async_copy(x_hbm.at[src_v], rows_v, sem)
            if with_deg:
                for j in range(KA // 16):
                    idx = dst_v[pl.ds(j * 16, 16)]
                    row = jax.lax.shift_right_logical(idx, 7)
                    col = jax.lax.bitwise_and(idx, 127)
                    plsc.addupdate_scatter(
                        deg_t, [row, col], jnp.ones((16,), jnp.float32))
            gat.wait()
            pltpu.sync_copy(rows_v, agg_s.at[dst_v], add=True)

        if with_deg:
            pltpu.sync_copy(deg_t, deg_s.at[idr_v], add=True)

        plsc.subcore_barrier()

        @pl.when(si == 0)
        def _drain():
            pltpu.sync_copy(agg_s, agg_out.at[pl.ds(ci * n_pad, n_pad), :])
            if with_deg:
                pltpu.sync_copy(
                    deg_s, deg_out.at[pl.ds(ci * nrow8, nrow8), :])

    cp = pltpu.CompilerParams()
    if with_deg and (
            "needs_layout_passes" in pltpu.CompilerParams.__dataclass_fields__):
        cp = dataclasses.replace(cp, needs_layout_passes=False)
    return pl.kernel(body, out_type=out_type, mesh=mesh,
                     scratch_types=scratch, compiler_params=cp)


def _tc_layer1(agg, deg, x, Wl, Wr, b):
    n, d = x.shape
    n_pad = agg.shape[0] // NC

    def body(agg_ref, deg_ref, x_ref, wl_ref, wr_ref, b_ref, h_ref):
        s = agg_ref[:n, :] + agg_ref[n_pad:n_pad + n, :]
        mean = s / jnp.maximum(deg_ref[...], 1.0)
        acc = jnp.dot(mean, wl_ref[...], preferred_element_type=jnp.float32)
        acc = acc + jnp.dot(x_ref[...], wr_ref[...],
                            preferred_element_type=jnp.float32)
        acc = acc + b_ref[...]
        h_ref[...] = jnp.maximum(acc, 0.0)

    return pl.pallas_call(
        body, out_shape=jax.ShapeDtypeStruct((n, d), jnp.float32),
    )(agg, deg, x, Wl, Wr, b.reshape(1, d))


def _tc_layer2(agg, deg, h, Wl, Wr, b):
    n, d = h.shape
    n_pad = agg.shape[0] // NC

    def body(agg_ref, deg_ref, h_ref, wl_ref, wr_ref, b_ref, o_ref, ls_ref):
        s = agg_ref[:n, :] + agg_ref[n_pad:n_pad + n, :]
        mean = s / jnp.maximum(deg_ref[...], 1.0)
        o = jnp.dot(mean, wl_ref[...], preferred_element_type=jnp.float32)
        o = o + jnp.dot(h_ref[...], wr_ref[...],
                        preferred_element_type=jnp.float32)
        o = o + b_ref[...]
        o_ref[...] = o
        m = jnp.max(o, axis=1, keepdims=True)
        lse = jnp.log(jnp.sum(jnp.exp(o - m), axis=1, keepdims=True)) + m
        ls_ref[...] = o - lse

    return pl.pallas_call(
        body,
        out_shape=(jax.ShapeDtypeStruct((n, d), jnp.float32),
                   jax.ShapeDtypeStruct((n, d), jnp.float32)),
    )(agg, deg, h, Wl, Wr, b.reshape(1, d))


def kernel(x, edge_index, W1l, W1r, b1, W2l, W2r, b2):
    n, d = x.shape
    e = edge_index.shape[1]
    ei = edge_index.astype(jnp.int32)
    src, dst = ei[0], ei[1]

    n_pad = -(-n // (NS * 8)) * (NS * 8)
    nrow8 = -(-(-(-n_pad // 128)) // 8) * 8
    zeros = jnp.zeros((n_pad, d), jnp.float32)
    iota_rows = jnp.arange(nrow8, dtype=jnp.int32)

    agg1, degp = _make_sc_agg(n_pad, d, e, True)(x, src, dst, zeros,
                                                 iota_rows)
    deg = (degp[:nrow8] + degp[nrow8:]).reshape(-1)[:n, None]
    h = _tc_layer1(agg1, deg, x, W1l, W1r, b1)
    agg2 = _make_sc_agg(n_pad, d, e)(h, src, dst, zeros)
    if isinstance(agg2, (list, tuple)):
        agg2 = agg2[0]
    out, ls = _tc_layer2(agg2, deg, h, W2l, W2r, b2)
    return (out, ls)
